# baseline (speedup 1.0000x reference)
import functools

import jax
import jax.numpy as jnp
from jax import lax
from jax.experimental import pallas as pl
from jax.experimental.pallas import tpu as pltpu, tpu_sc as plsc


@functools.lru_cache(maxsize=None)
def _build_router(batch: int, n_classes: int, n_paths: int):
    info = plsc.get_sparse_core_info()
    nc, ns, lanes = 1, info.num_subcores, info.num_lanes
    nw = nc * ns
    assert lanes == 16
    assert batch % (nw * lanes) == 0
    b_per_w = batch // nw
    out_per_w = b_per_w * n_paths
    n_iter = b_per_w // lanes

    mesh = plsc.VectorSubcoreMesh(
        core_axis_name="c", subcore_axis_name="s", num_cores=nc
    )

    @functools.partial(
        pl.kernel,
        mesh=mesh,
        out_type=jax.ShapeDtypeStruct((batch * n_paths,), jnp.float32),
        compiler_params=pltpu.CompilerParams(
            needs_layout_passes=False,
            skip_device_barrier=True,
            disable_semaphore_checks=True,
            disable_bounds_checks=True,
        ),
        scratch_types=[
            pltpu.VMEM((b_per_w,), jnp.int32),
            pltpu.VMEM((n_classes * n_paths,), jnp.float32),
            pltpu.VMEM((out_per_w,), jnp.float32),
            pltpu.SemaphoreType.DMA,
            pltpu.SemaphoreType.DMA,
        ],
    )
    def router(labels_hbm, rm_hbm, out_hbm, lbl_v, rm_v, out_v, sem_l, sem_r):
        wid = lax.axis_index("s") * nc + lax.axis_index("c")
        base = wid * b_per_w

        h_l = pltpu.async_copy(labels_hbm.at[pl.ds(base, b_per_w)], lbl_v, sem_l)
        h_r = pltpu.async_copy(rm_hbm, rm_v, sem_r)
        h_l.wait()
        h_r.wait()

        pos = lax.iota(jnp.int32, 16)
        npv = jnp.full((16,), n_paths, dtype=jnp.int32)
        pos_np = pos * npv
        rvs = [jnp.full((16,), r, dtype=jnp.int32) for r in range(n_paths)]

        for i in range(n_iter):
            lbl16 = lbl_v[pl.ds(i * lanes, lanes)]
            row_base = lbl16 * npv
            obase = jnp.full((16,), i * lanes * n_paths, dtype=jnp.int32)
            out_pos = obase + pos_np
            for r in range(n_paths):
                vals = plsc.load_gather(rm_v, [row_base + rvs[r]])
                plsc.store_scatter(out_v, [out_pos + rvs[r]], vals)

        pltpu.sync_copy(out_v, out_hbm.at[pl.ds(base * n_paths, out_per_w)])

    return router


def kernel(layer_input, labels, route_matrix, temperature, balance_coefficient):
    batch = labels.shape[0]
    n_classes, n_paths = route_matrix.shape
    router = _build_router(batch, n_classes, n_paths)
    out_flat = router(labels, route_matrix.reshape(-1))
    p_n_given_x = out_flat.reshape(batch, n_paths)
    return (p_n_given_x, p_n_given_x)


# speedup vs baseline: 1.0130x; 1.0130x over previous
import functools

import jax
import jax.numpy as jnp
from jax import lax
from jax.experimental import pallas as pl
from jax.experimental.pallas import tpu as pltpu, tpu_sc as plsc


@functools.lru_cache(maxsize=None)
def _build_router(batch: int, n_classes: int, n_paths: int):
    info = plsc.get_sparse_core_info()
    nc, ns, lanes = info.num_cores, info.num_subcores, info.num_lanes
    nw = nc * ns
    assert lanes == 16
    assert batch % (nw * lanes) == 0
    b_per_w = batch // nw
    out_per_w = b_per_w * n_paths
    n_iter = b_per_w // lanes

    mesh = plsc.VectorSubcoreMesh(
        core_axis_name="c", subcore_axis_name="s", num_cores=nc
    )

    @functools.partial(
        pl.kernel,
        mesh=mesh,
        out_type=jax.ShapeDtypeStruct((batch * n_paths,), jnp.float32),
        compiler_params=pltpu.CompilerParams(
            needs_layout_passes=False,
            skip_device_barrier=True,
            disable_semaphore_checks=True,
            disable_bounds_checks=True,
        ),
        scratch_types=[
            pltpu.VMEM((b_per_w,), jnp.int32),
            pltpu.VMEM((n_classes * n_paths,), jnp.float32),
            pltpu.VMEM((out_per_w,), jnp.float32),
            pltpu.SemaphoreType.DMA,
            pltpu.SemaphoreType.DMA,
            pltpu.SemaphoreType.DMA,
        ],
    )
    def router(labels_hbm, rm_hbm, out_hbm, lbl_v, rm_v, out_v, sem_l, sem_r, sem_o):
        wid = lax.axis_index("s") * nc + lax.axis_index("c")
        base = wid * b_per_w
        half = b_per_w // 2
        half_iters = n_iter // 2
        half_out = out_per_w // 2

        h_l0 = pltpu.async_copy(labels_hbm.at[pl.ds(base, half)], lbl_v.at[pl.ds(0, half)], sem_l)
        h_r = pltpu.async_copy(rm_hbm, rm_v, sem_r)
        h_l1 = pltpu.async_copy(
            labels_hbm.at[pl.ds(base + half, half)], lbl_v.at[pl.ds(half, half)], sem_l
        )
        h_l0.wait()
        h_r.wait()

        pos = lax.iota(jnp.int32, 16)
        npv = jnp.full((16,), n_paths, dtype=jnp.int32)
        pos_np = pos * npv
        rvs = [jnp.full((16,), r, dtype=jnp.int32) for r in range(n_paths)]

        def do_group(i):
            lbl16 = lbl_v[pl.ds(i * lanes, lanes)]
            row_base = lbl16 * npv
            obase = jnp.full((16,), i * lanes * n_paths, dtype=jnp.int32)
            out_pos = obase + pos_np
            for r in range(n_paths):
                vals = plsc.load_gather(rm_v, [row_base + rvs[r]])
                plsc.store_scatter(out_v, [out_pos + rvs[r]], vals)

        for i in range(half_iters):
            do_group(i)
        h_o0 = pltpu.async_copy(
            out_v.at[pl.ds(0, half_out)],
            out_hbm.at[pl.ds(base * n_paths, half_out)],
            sem_o,
        )
        h_l1.wait()
        for i in range(half_iters, n_iter):
            do_group(i)
        h_o1 = pltpu.async_copy(
            out_v.at[pl.ds(half_out, half_out)],
            out_hbm.at[pl.ds(base * n_paths + half_out, half_out)],
            sem_o,
        )
        h_o0.wait()
        h_o1.wait()

    return router


def kernel(layer_input, labels, route_matrix, temperature, balance_coefficient):
    batch = labels.shape[0]
    n_classes, n_paths = route_matrix.shape
    router = _build_router(batch, n_classes, n_paths)
    out_flat = router(labels, route_matrix.reshape(-1))
    p_n_given_x = out_flat.reshape(batch, n_paths)
    return (p_n_given_x, p_n_given_x)
